# baseline (speedup 1.0000x reference)
import functools

import jax
import jax.numpy as jnp
from jax import lax
from jax.experimental import pallas as pl
from jax.experimental.pallas import tpu as pltpu
from jax.experimental.pallas import tpu_sc as plsc

_VOCAB = 1000000
_D = 64
_B = 4096
_H = 200
_HB = _B // 2
_TOTAL = _B * _H
_NW = 32
_PER_W = _TOTAL // _NW
_C = 128
_NCHUNK = _PER_W // _C
_NB = 4
_K = 2
_L = 16

_T_COLS = 8192
_T_GRID = -(-_VOCAB // _T_COLS)


def _to_rows_body(t_ref, out_ref):
    eye2 = jnp.concatenate(
        [jnp.eye(_D, dtype=jnp.float32),
         jnp.zeros((_D, _D), dtype=jnp.float32)], axis=1)
    out_ref[...] = lax.dot_general(
        t_ref[...], eye2, (((0,), (0,)), ((), ())),
        preferred_element_type=jnp.float32,
    )


_to_rows = pl.pallas_call(
    _to_rows_body,
    grid=(_T_GRID,),
    in_specs=[pl.BlockSpec((_D, _T_COLS), lambda i: (0, i))],
    out_specs=pl.BlockSpec((_T_COLS, 2 * _D), lambda i: (i, 0)),
    out_shape=jax.ShapeDtypeStruct((_VOCAB, 2 * _D), jnp.float32),
)

_mesh = plsc.VectorSubcoreMesh(core_axis_name="c", subcore_axis_name="s")


@functools.partial(
    pl.kernel,
    out_type=jax.ShapeDtypeStruct((_H, _D, _B), jnp.float32),
    mesh=_mesh,
    scratch_types=[
        pltpu.VMEM((_PER_W,), jnp.int32),
        [pltpu.VMEM((_C, 2 * _D), jnp.float32) for _ in range(_NB)],
        [pltpu.VMEM((_D, _C), jnp.float32) for _ in range(2)],
        [pltpu.SemaphoreType.DMA for _ in range(_NB)],
        [pltpu.SemaphoreType.DMA for _ in range(2)],
    ],
    compiler_params=pltpu.CompilerParams(
        use_tc_tiling_on_sc=False, needs_layout_passes=False
    ),
)
def _gather(idx_hbm, table_hbm, out_hbm, idx_v, rows, tb, gsem, wsem):
    wid = lax.axis_index("s") * 2 + lax.axis_index("c")
    wbase = wid * _PER_W

    pltpu.sync_copy(idx_hbm.at[pl.ds(wbase, _PER_W)], idx_v)

    lanes = lax.iota(jnp.int32, _L)
    row_base = [lanes + g * _L for g in range(_C // _L)]

    def start_gather(chunk, b):
        pltpu.async_copy(
            table_hbm.at[idx_v.at[pl.ds(chunk * _C, _C)]], rows[b], gsem[b]
        )

    def out_slice(chunk):
        k0 = wbase + chunk * _C
        h = k0 // _B
        col0 = (k0 // _HB) % 2 * _HB + k0 % _HB
        return out_hbm.at[h, :, pl.ds(col0, _C)]

    for t in range(_K):
        start_gather(t, t)

    @pl.loop(0, _NCHUNK, step=_NB)
    def turn(t0):
        for b in range(_NB):
            t = t0 + b
            q = b % 2
            pa = (b + _K) % _NB

            @pl.when(t + _K < _NCHUNK)
            def _issue_ahead():
                start_gather(t + _K, pa)

            pltpu.make_async_copy(
                table_hbm.at[idx_v.at[pl.ds(t * _C, _C)]], rows[b], gsem[b]
            ).wait()

            @pl.when(t >= 2)
            def _drain_prev_writeback():
                pltpu.make_async_copy(tb[q], out_slice(t - 2), wsem[q]).wait()

            @pl.loop(0, _D)
            def _tpose(d):
                col = jnp.full((_L,), d, dtype=jnp.int32)
                for g in range(_C // _L):
                    v = plsc.load_gather(rows[b], [row_base[g], col])
                    tb[q][d, pl.ds(g * _L, _L)] = v

            pltpu.async_copy(tb[q], out_slice(t), wsem[q])

    for t in (_NCHUNK - 2, _NCHUNK - 1):
        pltpu.make_async_copy(tb[t % 2], out_slice(t), wsem[t % 2]).wait()


def kernel(indices, table):
    t_rows = _to_rows(table.T)
    idx_flat = indices.T.reshape(-1)
    o_phys = _gather(idx_flat, t_rows)
    return jnp.transpose(o_phys, (2, 0, 1))


# speedup vs baseline: 2.7964x; 2.7964x over previous
import functools

import jax
import jax.numpy as jnp
from jax import lax
from jax.experimental import pallas as pl
from jax.experimental.pallas import tpu as pltpu
from jax.experimental.pallas import tpu_sc as plsc

_VOCAB = 1000000
_D = 64
_B = 4096
_H = 200
_HB = _B // 2
_TOTAL = _B * _H
_NW = 32
_PER_W = _TOTAL // _NW
_C = 128
_NCHUNK = _PER_W // _C
_NB = 4
_K = 2

_T_COLS = 16384
_T_GRID = -(-_VOCAB // _T_COLS)


def _to_rows_body(t_ref, out_ref):
    eye2 = jnp.concatenate(
        [jnp.eye(_D, dtype=jnp.float32),
         jnp.zeros((_D, _D), dtype=jnp.float32)], axis=1)
    out_ref[...] = lax.dot_general(
        t_ref[...], eye2, (((0,), (0,)), ((), ())),
        preferred_element_type=jnp.float32,
    )


_to_rows = pl.pallas_call(
    _to_rows_body,
    grid=(_T_GRID,),
    in_specs=[pl.BlockSpec((_D, _T_COLS), lambda i: (0, i))],
    out_specs=pl.BlockSpec((_T_COLS, 2 * _D), lambda i: (i, 0)),
    out_shape=jax.ShapeDtypeStruct((_VOCAB, 2 * _D), jnp.float32),
)

_H_BLK = 8


def _to_out_body(x_ref, out_ref):
    eye = jnp.eye(_D, dtype=jnp.float32)
    for hh in range(_H_BLK):
        x = x_ref[pl.ds(hh * _HB, _HB), :]
        out_ref[hh, :, 0:_HB] = lax.dot_general(
            eye, x[:, 0:_D], (((1,), (1,)), ((), ())),
            preferred_element_type=jnp.float32,
        )
        out_ref[hh, :, _HB:_B] = lax.dot_general(
            eye, x[:, _D : 2 * _D], (((1,), (1,)), ((), ())),
            preferred_element_type=jnp.float32,
        )


_to_out = pl.pallas_call(
    _to_out_body,
    grid=(_H // _H_BLK,),
    in_specs=[pl.BlockSpec((_H_BLK * _HB, 2 * _D), lambda g: (g, 0))],
    out_specs=pl.BlockSpec((_H_BLK, _D, _B), lambda g: (g, 0, 0)),
    out_shape=jax.ShapeDtypeStruct((_H, _D, _B), jnp.float32),
)

_mesh = plsc.VectorSubcoreMesh(core_axis_name="c", subcore_axis_name="s")


@functools.partial(
    pl.kernel,
    out_type=jax.ShapeDtypeStruct((_TOTAL // 2, 2 * _D), jnp.float32),
    mesh=_mesh,
    scratch_types=[
        pltpu.VMEM((_PER_W,), jnp.int32),
        [pltpu.VMEM((_C, 2 * _D), jnp.float32) for _ in range(_NB)],
        [pltpu.SemaphoreType.DMA for _ in range(_NB)],
        [pltpu.SemaphoreType.DMA for _ in range(_NB)],
    ],
    compiler_params=pltpu.CompilerParams(use_tc_tiling_on_sc=False),
)
def _gather(idx_hbm, table_hbm, out_hbm, idx_v, rows, gsem, wsem):
    wid = lax.axis_index("s") * 2 + lax.axis_index("c")
    wbase = wid * _PER_W

    pltpu.sync_copy(idx_hbm.at[pl.ds(wbase, _PER_W)], idx_v)

    def start_gather(chunk, b):
        pltpu.async_copy(
            table_hbm.at[idx_v.at[pl.ds(chunk * _C, _C)]], rows[b], gsem[b]
        )

    def wb_pair(chunk):
        k0 = wbase + chunk * _C
        h = k0 // _B
        p = (k0 // _HB) % 2
        r0 = h * _HB + k0 % _HB
        return (
            lambda b: rows[b].at[:, pl.ds(0, _D)],
            lambda: out_hbm.at[pl.ds(r0, _C), pl.ds(p * _D, _D)],
        )

    for t in range(_K):
        start_gather(t, t)

    @pl.loop(0, _NCHUNK, step=_NB)
    def turn(t0):
        for b in range(_NB):
            t = t0 + b
            pa = (b + _K) % _NB

            @pl.when(t + _K < _NCHUNK)
            def _issue_ahead():
                @pl.when(t + _K >= _NB)
                def _drain_prev_writeback():
                    src, dst = wb_pair(t + _K - _NB)
                    pltpu.make_async_copy(src(pa), dst(), wsem[pa]).wait()

                start_gather(t + _K, pa)

            pltpu.make_async_copy(
                table_hbm.at[idx_v.at[pl.ds(t * _C, _C)]], rows[b], gsem[b]
            ).wait()
            src, dst = wb_pair(t)
            pltpu.async_copy(src(b), dst(), wsem[b])

    for b in range(_NB):
        src, dst = wb_pair(_NCHUNK - _NB + b)
        pltpu.make_async_copy(src(b), dst(), wsem[b]).wait()


def kernel(indices, table):
    t_rows = _to_rows(table.T)
    idx_flat = indices.T.reshape(-1)
    o_pairs = _gather(idx_flat, t_rows)
    o_phys = _to_out(o_pairs)
    return jnp.transpose(o_phys, (2, 0, 1))
